# Initial kernel scaffold; baseline (speedup 1.0000x reference)
import jax
import jax.numpy as jnp
from jax.experimental import pallas as pl


def kernel(data, lengths, table):
    raise NotImplementedError("write your pallas kernel here")


# speedup vs baseline: 16.4025x; 16.4025x over previous
import functools

import jax
import jax.numpy as jnp
from jax import lax
from jax.experimental import pallas as pl
from jax.experimental.pallas import tpu as pltpu
from jax.experimental.pallas import tpu_sc as plsc

B = 16384
L = 200
D = 32
NC = 2
NS = 16
NW = NC * NS
RPW = B // NW
C = 8
NCHUNK = RPW // C
IC = C * L


def _pool_body(data_hbm, table_hbm, out_hbm,
               idx0, idx1, rows0, rows1, out_v, sem0, sem1):
    wid = lax.axis_index("s") * NC + lax.axis_index("c")
    base = wid * RPW

    idx_bufs = (idx0, idx1)
    rows_bufs = (rows0, rows1)
    sems = (sem0, sem1)

    def start_gather(g, b):
        pltpu.sync_copy(data_hbm.at[pl.ds((base + g * C) * L, IC)], idx_bufs[b])
        pltpu.async_copy(table_hbm.at[idx_bufs[b]], rows_bufs[b], sems[b])

    def wait_gather(b):
        pltpu.make_async_copy(table_hbm.at[idx_bufs[b]], rows_bufs[b],
                              sems[b]).wait()

    def accumulate(rows, g):
        for c in range(C):
            def body(j, accs, c=c):
                a0, a1, b0, b1 = accs
                r = c * L + 2 * j
                a0 = a0 + rows[r, pl.ds(0, 16)]
                a1 = a1 + rows[r, pl.ds(16, 16)]
                b0 = b0 + rows[r + 1, pl.ds(0, 16)]
                b1 = b1 + rows[r + 1, pl.ds(16, 16)]
                return (a0, a1, b0, b1)

            z = jnp.zeros((16,), jnp.float32)
            a0, a1, b0, b1 = lax.fori_loop(0, L // 2, body, (z, z, z, z),
                                           unroll=4)
            out_v[c, pl.ds(0, 16)] = a0 + b0
            out_v[c, pl.ds(16, 16)] = a1 + b1
        pltpu.sync_copy(out_v, out_hbm.at[pl.ds(base + g * C, C)])

    start_gather(0, 0)
    start_gather(1, 1)

    def outer(g2, carry):
        for b in range(2):
            g = 2 * g2 + b
            wait_gather(b)
            accumulate(rows_bufs[b], g)
            start_gather(g + 2, b)
        return carry

    lax.fori_loop(0, NCHUNK // 2 - 1, outer, 0)

    for b in range(2):
        g = NCHUNK - 2 + b
        wait_gather(b)
        accumulate(rows_bufs[b], g)


@functools.partial(jax.jit)
def kernel(data, lengths, table):
    del lengths
    data_flat = data.reshape(-1).astype(jnp.int32)
    mesh = plsc.VectorSubcoreMesh(core_axis_name="c", subcore_axis_name="s")
    run = pl.kernel(
        _pool_body,
        out_type=jax.ShapeDtypeStruct((B, D), jnp.float32),
        mesh=mesh,
        compiler_params=pltpu.CompilerParams(use_tc_tiling_on_sc=False),
        scratch_types=[
            pltpu.VMEM((IC,), jnp.int32),
            pltpu.VMEM((IC,), jnp.int32),
            pltpu.VMEM((IC, D), jnp.float32),
            pltpu.VMEM((IC, D), jnp.float32),
            pltpu.VMEM((C, D), jnp.float32),
            pltpu.SemaphoreType.DMA,
            pltpu.SemaphoreType.DMA,
        ],
    )
    return run(data_flat, table)
